# Initial kernel scaffold; baseline (speedup 1.0000x reference)
import functools

import jax
import jax.numpy as jnp
from jax import lax
from jax.experimental import pallas as pl
from jax.experimental.pallas import tpu as pltpu
from jax.experimental.pallas import tpu_sc as plsc

EMBEDDING_DIM = 64


@functools.cache
def _make_sc_gather(B: int, D: int, n_workers: int, chunk: int):
    b_per_w = B // n_workers
    n_chunks = b_per_w // chunk
    mesh = plsc.VectorSubcoreMesh(core_axis_name="c", subcore_axis_name="s")

    @functools.partial(
        pl.kernel,
        mesh=mesh,
        out_type=jax.ShapeDtypeStruct((B, D), jnp.float32),
        scratch_types=[
            pltpu.VMEM((chunk,), jnp.int32),
            pltpu.VMEM((chunk, D), jnp.float32),
            pltpu.SemaphoreType.DMA,
        ],
    )
    def k(table_hbm, idx_hbm, out_hbm, idx_v, rows_v, sem):
        wid = lax.axis_index("s") * 2 + lax.axis_index("c")
        base0 = wid * b_per_w

        def body(c, carry):
            base = base0 + c * chunk
            pltpu.sync_copy(idx_hbm.at[pl.ds(base, chunk)], idx_v)
            pltpu.async_copy(table_hbm.at[idx_v], rows_v, sem).wait()
            pltpu.sync_copy(rows_v, out_hbm.at[pl.ds(base, chunk)])
            return carry

        lax.fori_loop(0, n_chunks, body, 0)

    return k


def kernel(x, W):
    B0, T = x.shape
    B = B0 * T
    flat_idx = x.reshape(B).astype(jnp.int32)
    gather = _make_sc_gather(B, EMBEDDING_DIM, 32, 512)
    out = gather(W, flat_idx)
    return out.reshape(B0, T, EMBEDDING_DIM)


# speedup vs baseline: 1.7970x; 1.7970x over previous
import functools

import jax
import jax.numpy as jnp
from jax import lax
from jax.experimental import pallas as pl
from jax.experimental.pallas import tpu as pltpu
from jax.experimental.pallas import tpu_sc as plsc

EMBEDDING_DIM = 64


@functools.cache
def _make_sc_gather(B: int, D: int, n_workers: int, chunk: int):
    b_per_w = B // n_workers
    n_chunks = b_per_w // chunk
    mesh = plsc.VectorSubcoreMesh(core_axis_name="c", subcore_axis_name="s")

    @functools.partial(
        pl.kernel,
        mesh=mesh,
        compiler_params=pltpu.CompilerParams(use_tc_tiling_on_sc=False),
        out_type=jax.ShapeDtypeStruct((B, D), jnp.float32),
        scratch_types=[
            pltpu.VMEM((chunk,), jnp.int32),
            pltpu.VMEM((chunk, D), jnp.float32),
            pltpu.SemaphoreType.DMA,
        ],
    )
    def k(table_hbm, idx_hbm, out_hbm, idx_v, rows_v, sem):
        wid = lax.axis_index("s") * 2 + lax.axis_index("c")
        base0 = wid * b_per_w

        def body(c, carry):
            base = base0 + c * chunk
            pltpu.sync_copy(idx_hbm.at[pl.ds(base, chunk)], idx_v)
            pltpu.async_copy(table_hbm.at[idx_v], rows_v, sem).wait()
            pltpu.sync_copy(rows_v, out_hbm.at[pl.ds(base, chunk)])
            return carry

        lax.fori_loop(0, n_chunks, body, 0)

    return k


def kernel(x, W):
    B0, T = x.shape
    B = B0 * T
    flat_idx = x.reshape(B).astype(jnp.int32)
    gather = _make_sc_gather(B, EMBEDDING_DIM, 32, 512)
    out = gather(W, flat_idx)
    return out.reshape(B0, T, EMBEDDING_DIM)


# speedup vs baseline: 1.8745x; 1.0431x over previous
import functools

import jax
import jax.numpy as jnp
from jax import lax
from jax.experimental import pallas as pl
from jax.experimental.pallas import tpu as pltpu
from jax.experimental.pallas import tpu_sc as plsc

EMBEDDING_DIM = 64


@functools.cache
def _make_sc_gather(B: int, D: int, n_workers: int, chunk: int, nbuf: int):
    b_per_w = B // n_workers
    n_chunks = b_per_w // chunk
    n_rounds = (n_chunks + nbuf - 1) // nbuf
    mesh = plsc.VectorSubcoreMesh(core_axis_name="c", subcore_axis_name="s")

    @functools.partial(
        pl.kernel,
        mesh=mesh,
        compiler_params=pltpu.CompilerParams(use_tc_tiling_on_sc=False),
        out_type=jax.ShapeDtypeStruct((B, D), jnp.float32),
        scratch_types=[
            pltpu.VMEM((b_per_w,), jnp.int32),
            pltpu.VMEM((nbuf, chunk, D), jnp.float32),
            pltpu.SemaphoreType.DMA((nbuf,)),
            pltpu.SemaphoreType.DMA((nbuf,)),
        ],
    )
    def k(table_hbm, idx_hbm, out_hbm, idx_v, rows_v, gsem, osem):
        wid = lax.axis_index("s") * 2 + lax.axis_index("c")
        base0 = wid * b_per_w
        pltpu.sync_copy(idx_hbm.at[pl.ds(base0, b_per_w)], idx_v)

        def gather(i, b):
            off = pl.multiple_of(i * chunk, chunk)
            return pltpu.make_async_copy(
                table_hbm.at[idx_v.at[pl.ds(off, chunk)]], rows_v.at[b], gsem.at[b]
            )

        def write(i, b):
            off = pl.multiple_of(base0 + i * chunk, chunk)
            return pltpu.make_async_copy(
                rows_v.at[b], out_hbm.at[pl.ds(off, chunk)], osem.at[b]
            )

        for b in range(nbuf):
            gather(b, b).start()

        def round_body(r, carry):
            for b in range(nbuf):
                i = r * nbuf + b

                @pl.when(i < n_chunks)
                def _():
                    gather(i, b).wait()
                    write(i, b).start()
                    nxt = i + nbuf

                    @pl.when(nxt < n_chunks)
                    def _():
                        write(i, b).wait()
                        gather(nxt, b).start()

            return carry

        lax.fori_loop(0, n_rounds, round_body, 0)

        for b in range(nbuf):
            last_i = ((n_chunks - 1 - b) // nbuf) * nbuf + b
            write(last_i, b).wait()

    return k


def kernel(x, W):
    B0, T = x.shape
    B = B0 * T
    flat_idx = x.reshape(B).astype(jnp.int32)
    gather = _make_sc_gather(B, EMBEDDING_DIM, 32, 512, 3)
    out = gather(W, flat_idx)
    return out.reshape(B0, T, EMBEDDING_DIM)
